# baseline (speedup 1.0000x reference)
import functools

import jax
import jax.numpy as jnp
from jax import lax
from jax.experimental import pallas as pl
from jax.experimental.pallas import tpu as pltpu


def _drnn_body(x_ref, wd_ref, we_ref, b_ref, o_ref,
               p0_ref, e0_ref, p1_ref, e1_ref, *, H, S, CP, n_sub):
    C = CP // 8
    M = H * S

    p0_ref[...] = jnp.zeros_like(p0_ref)
    e0_ref[...] = jnp.zeros_like(e0_ref)
    p1_ref[...] = jnp.zeros_like(p1_ref)
    e1_ref[...] = jnp.zeros_like(e1_ref)

    def conv3x3(hb, p_ref, e_ref, li):
        p_ref[1:H + 1] = hb
        left = hb[:, 0:S - 1, CP - C:CP]
        right = hb[:, 1:S, 0:C]
        e_ref[1:H, 1:S, 0:C] = left[0:H - 1]
        e_ref[1:H, 0:S - 1, C:2 * C] = right[0:H - 1]
        e_ref[0:1, :, 0:2 * C] = jnp.zeros((1, S, 2 * C), jnp.bfloat16)
        e_ref[:, 1:S, 2 * C:3 * C] = left
        e_ref[:, 0:S - 1, 3 * C:4 * C] = right
        e_ref[0:H - 1, 1:S, 4 * C:5 * C] = left[1:H]
        e_ref[0:H - 1, 0:S - 1, 5 * C:6 * C] = right[1:H]
        e_ref[H - 1:H, :, 4 * C:6 * C] = jnp.zeros((1, S, 2 * C), jnp.bfloat16)
        acc = jnp.dot(p_ref[0:H].reshape(M, CP), wd_ref[li, 0],
                      preferred_element_type=jnp.float32)
        acc = acc + jnp.dot(p_ref[1:H + 1].reshape(M, CP), wd_ref[li, 1],
                            preferred_element_type=jnp.float32)
        acc = acc + jnp.dot(p_ref[2:H + 2].reshape(M, CP), wd_ref[li, 2],
                            preferred_element_type=jnp.float32)
        acc = acc + jnp.dot(e_ref[...].reshape(M, 6 * C), we_ref[li],
                            preferred_element_type=jnp.float32)
        acc = acc + b_ref[li]
        return acc.reshape(H, S, CP)

    def conv2(hv0, hv1, li):
        a0 = conv3x3(hv0, p0_ref, e0_ref, li)
        a1 = conv3x3(hv1, p1_ref, e1_ref, li)
        return a0, a1

    def relu2(a, b):
        return (jnp.maximum(a, 0.0).astype(jnp.bfloat16),
                jnp.maximum(b, 0.0).astype(jnp.bfloat16))

    def relu2_res(a, b, ra, rb):
        return ((jnp.maximum(a, 0.0) + ra.astype(jnp.float32))
                .astype(jnp.bfloat16),
                (jnp.maximum(b, 0.0) + rb.astype(jnp.float32))
                .astype(jnp.bfloat16))

    def subnet(si, carry):
        h0, h1 = carry
        base = si * 7
        a1, b1 = relu2(*conv2(h0.astype(jnp.bfloat16),
                              h1.astype(jnp.bfloat16), base + 0))
        a2, b2 = relu2(*conv2(a1, b1, base + 1))
        a3, b3_ = relu2(*conv2(a2, b2, base + 2))
        a4, b4 = relu2_res(*conv2(a3, b3_, base + 3), a3, b3_)
        a5, b5 = relu2_res(*conv2(a4, b4, base + 4), a2, b2)
        a6, b6 = relu2_res(*conv2(a5, b5, base + 5), a1, b1)
        a7, b7 = conv2(a6, b6, base + 6)
        return a7 + h0, b7 + h1

    h0, h1 = lax.fori_loop(
        0, n_sub, subnet,
        (x_ref[0].astype(jnp.float32), x_ref[1].astype(jnp.float32)))
    o_ref[0] = h0
    o_ref[1] = h1


def _pack_weights(packed_w, packed_b, L, C):
    CP = 8 * C
    taps = packed_w.reshape(L, 3, 3, C, C)
    q = jnp.arange(8)
    mask = (q[None, :, None] == q[None, None, :] +
            jnp.arange(3)[:, None, None] - 1).astype(jnp.float32)
    wd = jnp.einsum('xqp,lyxio->lyqipo', mask, taps)
    wd = wd.reshape(L, 3, CP, CP).astype(jnp.bfloat16)
    we = jnp.zeros((L, 3, 2 * C, CP), jnp.float32)
    we = we.at[:, :, 0:C, 0:C].set(taps[:, :, 0])
    we = we.at[:, :, C:2 * C, CP - C:CP].set(taps[:, :, 2])
    we = we.reshape(L, 6 * C, CP).astype(jnp.bfloat16)
    b3 = jnp.tile(packed_b, (1, 8, 1)).reshape(L, 1, CP)
    return wd, we, b3


@jax.jit
def kernel(x, packed_w, packed_b):
    N, H, W, Cin = x.shape
    C = packed_w.shape[-1]
    L = packed_b.shape[0]
    n_sub = L // 7
    S = W // 8
    CP = 8 * C

    wd, we, b3 = _pack_weights(packed_w, packed_b, L, C)
    xp = jnp.pad(x, ((0, 0), (0, 0), (0, 0), (0, C - Cin)))
    xp = xp.reshape(N, H, S, CP)

    body = functools.partial(_drnn_body, H=H, S=S, CP=CP, n_sub=n_sub)
    out = pl.pallas_call(
        body,
        out_shape=jax.ShapeDtypeStruct((N, H, S, CP), x.dtype),
        grid_spec=pltpu.PrefetchScalarGridSpec(
            num_scalar_prefetch=0,
            grid=(N // 2,),
            in_specs=[
                pl.BlockSpec((2, H, S, CP), lambda n: (n, 0, 0, 0)),
                pl.BlockSpec(wd.shape, lambda n: (0, 0, 0, 0)),
                pl.BlockSpec(we.shape, lambda n: (0, 0, 0)),
                pl.BlockSpec(b3.shape, lambda n: (0, 0, 0)),
            ],
            out_specs=pl.BlockSpec((2, H, S, CP), lambda n: (n, 0, 0, 0)),
            scratch_shapes=[
                pltpu.VMEM((H + 2, S, CP), jnp.bfloat16),
                pltpu.VMEM((H, S, 6 * C), jnp.bfloat16),
                pltpu.VMEM((H + 2, S, CP), jnp.bfloat16),
                pltpu.VMEM((H, S, 6 * C), jnp.bfloat16),
            ],
        ),
        compiler_params=pltpu.CompilerParams(
            dimension_semantics=("parallel",)),
    )(xp, wd, we, b3)
    return out.reshape(N, H, W, C)[..., :Cin]


# speedup vs baseline: 1.1430x; 1.1430x over previous
import functools

import jax
import jax.numpy as jnp
from jax import lax
from jax.experimental import pallas as pl
from jax.experimental.pallas import tpu as pltpu

_IPP = 4


def _drnn_body(x_ref, wd_ref, we_ref, b_ref, o_ref, *scratch,
               H, S, CP, n_sub):
    C = CP // 8
    M = H * S
    p_refs = scratch[0::2]
    e_refs = scratch[1::2]

    for r in scratch:
        r[...] = jnp.zeros_like(r)

    def conv3x3(h_val, p_ref, e_ref, li, relu):
        hb = h_val.astype(jnp.bfloat16)
        p_ref[1:H + 1] = hb
        left = hb[:, 0:S - 1, CP - C:CP]
        right = hb[:, 1:S, 0:C]
        e_ref[1:H, 1:S, 0:C] = left[0:H - 1]
        e_ref[1:H, 0:S - 1, C:2 * C] = right[0:H - 1]
        e_ref[0:1, :, 0:2 * C] = jnp.zeros((1, S, 2 * C), jnp.bfloat16)
        e_ref[:, 1:S, 2 * C:3 * C] = left
        e_ref[:, 0:S - 1, 3 * C:4 * C] = right
        e_ref[0:H - 1, 1:S, 4 * C:5 * C] = left[1:H]
        e_ref[0:H - 1, 0:S - 1, 5 * C:6 * C] = right[1:H]
        e_ref[H - 1:H, :, 4 * C:6 * C] = jnp.zeros((1, S, 2 * C), jnp.bfloat16)
        acc = jnp.dot(p_ref[0:H].reshape(M, CP), wd_ref[li, 0],
                      preferred_element_type=jnp.float32)
        acc = acc + jnp.dot(p_ref[1:H + 1].reshape(M, CP), wd_ref[li, 1],
                            preferred_element_type=jnp.float32)
        acc = acc + jnp.dot(p_ref[2:H + 2].reshape(M, CP), wd_ref[li, 2],
                            preferred_element_type=jnp.float32)
        acc = acc + jnp.dot(e_ref[...].reshape(M, 6 * C), we_ref[li],
                            preferred_element_type=jnp.float32)
        acc = acc + b_ref[li]
        if relu:
            acc = jnp.maximum(acc, 0.0)
        return acc.reshape(H, S, CP)

    def convs(hs, li, relu):
        return tuple(conv3x3(h, p_refs[i], e_refs[i], li, relu)
                     for i, h in enumerate(hs))

    def add(xs, ys):
        return tuple(xv + yv for xv, yv in zip(xs, ys))

    def subnet(si, hs):
        base = si * 7
        a1 = convs(hs, base + 0, True)
        a2 = convs(a1, base + 1, True)
        a3 = convs(a2, base + 2, True)
        a4 = add(convs(a3, base + 3, True), a3)
        a5 = add(convs(a4, base + 4, True), a2)
        a6 = add(convs(a5, base + 5, True), a1)
        return add(convs(a6, base + 6, False), hs)

    hs = lax.fori_loop(
        0, n_sub, subnet,
        tuple(x_ref[i].astype(jnp.float32) for i in range(len(p_refs))))
    for i, h in enumerate(hs):
        o_ref[i] = h


def _pack_weights(packed_w, packed_b, L, C):
    CP = 8 * C
    taps = packed_w.reshape(L, 3, 3, C, C)
    q = jnp.arange(8)
    mask = (q[None, :, None] == q[None, None, :] +
            jnp.arange(3)[:, None, None] - 1).astype(jnp.float32)
    wd = jnp.einsum('xqp,lyxio->lyqipo', mask, taps)
    wd = wd.reshape(L, 3, CP, CP).astype(jnp.bfloat16)
    we = jnp.zeros((L, 3, 2 * C, CP), jnp.float32)
    we = we.at[:, :, 0:C, 0:C].set(taps[:, :, 0])
    we = we.at[:, :, C:2 * C, CP - C:CP].set(taps[:, :, 2])
    we = we.reshape(L, 6 * C, CP).astype(jnp.bfloat16)
    b3 = jnp.tile(packed_b, (1, 8, 1)).reshape(L, 1, CP)
    return wd, we, b3


@jax.jit
def kernel(x, packed_w, packed_b):
    N, H, W, Cin = x.shape
    C = packed_w.shape[-1]
    L = packed_b.shape[0]
    n_sub = L // 7
    S = W // 8
    CP = 8 * C
    ipp = _IPP if N % _IPP == 0 else 1

    wd, we, b3 = _pack_weights(packed_w, packed_b, L, C)
    xp = jnp.pad(x, ((0, 0), (0, 0), (0, 0), (0, C - Cin)))
    xp = xp.reshape(N, H, S, CP)

    body = functools.partial(_drnn_body, H=H, S=S, CP=CP, n_sub=n_sub)
    out = pl.pallas_call(
        body,
        out_shape=jax.ShapeDtypeStruct((N, H, S, CP), x.dtype),
        grid_spec=pltpu.PrefetchScalarGridSpec(
            num_scalar_prefetch=0,
            grid=(N // ipp,),
            in_specs=[
                pl.BlockSpec((ipp, H, S, CP), lambda n: (n, 0, 0, 0)),
                pl.BlockSpec(wd.shape, lambda n: (0, 0, 0, 0)),
                pl.BlockSpec(we.shape, lambda n: (0, 0, 0)),
                pl.BlockSpec(b3.shape, lambda n: (0, 0, 0)),
            ],
            out_specs=pl.BlockSpec((ipp, H, S, CP), lambda n: (n, 0, 0, 0)),
            scratch_shapes=[
                pltpu.VMEM((H + 2, S, CP), jnp.bfloat16) if i % 2 == 0
                else pltpu.VMEM((H, S, 6 * C), jnp.bfloat16)
                for i in range(2 * ipp)
            ],
        ),
        compiler_params=pltpu.CompilerParams(
            dimension_semantics=("parallel",)),
    )(xp, wd, we, b3)
    return out.reshape(N, H, W, C)[..., :Cin]


# speedup vs baseline: 1.1597x; 1.0146x over previous
import functools

import jax
import jax.numpy as jnp
from jax import lax
from jax.experimental import pallas as pl
from jax.experimental.pallas import tpu as pltpu

_IPP = 4


def _drnn_body(x_ref, wd_ref, we_ref, b_ref, o_ref, *scratch,
               H, S, CP, n_sub):
    C = CP // 8
    M = H * S
    p_refs = scratch[0::2]
    e_refs = scratch[1::2]

    for r in scratch:
        r[...] = jnp.zeros_like(r)

    def conv3x3(h_val, p_ref, e_ref, li, relu):
        hb = h_val.astype(jnp.bfloat16)
        p_ref[1:H + 1] = hb
        left = hb[:, 0:S - 1, CP - C:CP]
        right = hb[:, 1:S, 0:C]
        e_ref[1:H, 1:S, 0:C] = left[0:H - 1]
        e_ref[1:H, 0:S - 1, C:2 * C] = right[0:H - 1]
        e_ref[0:1, :, 0:2 * C] = jnp.zeros((1, S, 2 * C), jnp.bfloat16)
        e_ref[:, 1:S, 2 * C:3 * C] = left
        e_ref[:, 0:S - 1, 3 * C:4 * C] = right
        e_ref[0:H - 1, 1:S, 4 * C:5 * C] = left[1:H]
        e_ref[0:H - 1, 0:S - 1, 5 * C:6 * C] = right[1:H]
        e_ref[H - 1:H, :, 4 * C:6 * C] = jnp.zeros((1, S, 2 * C), jnp.bfloat16)
        acc = jnp.dot(p_ref[0:H].reshape(M, CP), wd_ref[li, 0],
                      preferred_element_type=jnp.float32)
        acc = acc + jnp.dot(p_ref[1:H + 1].reshape(M, CP), wd_ref[li, 1],
                            preferred_element_type=jnp.float32)
        acc = acc + jnp.dot(p_ref[2:H + 2].reshape(M, CP), wd_ref[li, 2],
                            preferred_element_type=jnp.float32)
        acc = acc + jnp.dot(e_ref[...].reshape(M, 6 * C), we_ref[li],
                            preferred_element_type=jnp.float32)
        acc = acc + b_ref[li]
        if relu:
            acc = jnp.maximum(acc, 0.0)
        return acc.reshape(H, S, CP)

    def convs(hs, li, relu):
        return tuple(conv3x3(h, p_refs[i], e_refs[i], li, relu)
                     for i, h in enumerate(hs))

    def add(xs, ys):
        return tuple(xv + yv for xv, yv in zip(xs, ys))

    def subnet(si, hs):
        base = si * 7
        a1 = convs(hs, base + 0, True)
        a2 = convs(a1, base + 1, True)
        a3 = convs(a2, base + 2, True)
        a4 = add(convs(a3, base + 3, True), a3)
        a5 = add(convs(a4, base + 4, True), a2)
        a6 = add(convs(a5, base + 5, True), a1)
        return add(convs(a6, base + 6, False), hs)

    hs = lax.fori_loop(
        0, n_sub, subnet,
        tuple(x_ref[i].astype(jnp.float32) for i in range(len(p_refs))))
    for i, h in enumerate(hs):
        o_ref[i] = h


def _pack_weights(packed_w, packed_b, L, C):
    CP = 8 * C
    taps = packed_w.reshape(L, 3, 3, C, C)
    q = jnp.arange(8)
    mask = (q[None, :, None] == q[None, None, :] +
            jnp.arange(3)[:, None, None] - 1).astype(jnp.float32)
    wd = jnp.einsum('xqp,lyxio->lyqipo', mask, taps)
    wd = wd.reshape(L, 3, CP, CP).astype(jnp.bfloat16)
    we = jnp.zeros((L, 3, 2 * C, CP), jnp.float32)
    we = we.at[:, :, 0:C, 0:C].set(taps[:, :, 0])
    we = we.at[:, :, C:2 * C, CP - C:CP].set(taps[:, :, 2])
    we = we.reshape(L, 6 * C, CP).astype(jnp.bfloat16)
    b3 = jnp.tile(packed_b, (1, 8, 1)).reshape(L, 1, CP)
    return wd, we, b3


@jax.jit
def kernel(x, packed_w, packed_b):
    N, H, W, Cin = x.shape
    C = packed_w.shape[-1]
    L = packed_b.shape[0]
    n_sub = L // 7
    S = W // 8
    CP = 8 * C
    ipp = _IPP if N % _IPP == 0 else 1

    wd, we, b3 = _pack_weights(packed_w, packed_b, L, C)
    xp = jnp.pad(x, ((0, 0), (0, 0), (0, 0), (0, C - Cin)))
    xp = xp.reshape(N, H, S, CP).astype(jnp.bfloat16)

    body = functools.partial(_drnn_body, H=H, S=S, CP=CP, n_sub=n_sub)
    out = pl.pallas_call(
        body,
        out_shape=jax.ShapeDtypeStruct((N, H, S, CP), x.dtype),
        grid_spec=pltpu.PrefetchScalarGridSpec(
            num_scalar_prefetch=0,
            grid=(N // ipp,),
            in_specs=[
                pl.BlockSpec((ipp, H, S, CP), lambda n: (n, 0, 0, 0)),
                pl.BlockSpec(wd.shape, lambda n: (0, 0, 0, 0)),
                pl.BlockSpec(we.shape, lambda n: (0, 0, 0)),
                pl.BlockSpec(b3.shape, lambda n: (0, 0, 0)),
            ],
            out_specs=pl.BlockSpec((ipp, H, S, CP), lambda n: (n, 0, 0, 0)),
            scratch_shapes=[
                pltpu.VMEM((H + 2, S, CP), jnp.bfloat16) if i % 2 == 0
                else pltpu.VMEM((H, S, 6 * C), jnp.bfloat16)
                for i in range(2 * ipp)
            ],
        ),
        compiler_params=pltpu.CompilerParams(
            dimension_semantics=("parallel",)),
    )(xp, wd, we, b3)
    return out.reshape(N, H, W, C)[..., :Cin]


# speedup vs baseline: 1.1846x; 1.0215x over previous
import functools

import jax
import jax.numpy as jnp
from jax import lax
from jax.experimental import pallas as pl
from jax.experimental.pallas import tpu as pltpu

_IPP = 4


def _drnn_body(x_ref, wd_ref, we_ref, b_ref, o_ref, *scratch,
               H, S, CP, n_sub):
    C = CP // 8
    M = H * S
    e_refs = scratch

    for r in scratch:
        r[...] = jnp.zeros_like(r)

    zrow = jnp.zeros((1, S, CP), jnp.bfloat16)

    def conv3x3(h_val, e_ref, li, relu):
        hb = h_val.astype(jnp.bfloat16)
        left = hb[:, 0:S - 1, CP - C:CP]
        right = hb[:, 1:S, 0:C]
        e_ref[1:H, 1:S, 0:C] = left[0:H - 1]
        e_ref[1:H, 0:S - 1, C:2 * C] = right[0:H - 1]
        e_ref[:, 1:S, 2 * C:3 * C] = left
        e_ref[:, 0:S - 1, 3 * C:4 * C] = right
        e_ref[0:H - 1, 1:S, 4 * C:5 * C] = left[1:H]
        e_ref[0:H - 1, 0:S - 1, 5 * C:6 * C] = right[1:H]
        x0 = jnp.concatenate([zrow, hb[0:H - 1]], axis=0).reshape(M, CP)
        x2 = jnp.concatenate([hb[1:H], zrow], axis=0).reshape(M, CP)
        acc = jnp.dot(x0, wd_ref[li, 0], preferred_element_type=jnp.float32)
        acc = acc + jnp.dot(hb.reshape(M, CP), wd_ref[li, 1],
                            preferred_element_type=jnp.float32)
        acc = acc + jnp.dot(x2, wd_ref[li, 2],
                            preferred_element_type=jnp.float32)
        acc = acc + jnp.dot(e_ref[...].reshape(M, 6 * C), we_ref[li],
                            preferred_element_type=jnp.float32)
        acc = acc + b_ref[li]
        if relu:
            acc = jnp.maximum(acc, 0.0)
        return acc.reshape(H, S, CP)

    def convs(hs, li, relu):
        return tuple(conv3x3(h, e_refs[i], li, relu)
                     for i, h in enumerate(hs))

    def add(xs, ys):
        return tuple(xv + yv for xv, yv in zip(xs, ys))

    def subnet(si, hs):
        base = si * 7
        a1 = convs(hs, base + 0, True)
        a2 = convs(a1, base + 1, True)
        a3 = convs(a2, base + 2, True)
        a4 = add(convs(a3, base + 3, True), a3)
        a5 = add(convs(a4, base + 4, True), a2)
        a6 = add(convs(a5, base + 5, True), a1)
        return add(convs(a6, base + 6, False), hs)

    hs = lax.fori_loop(
        0, n_sub, subnet,
        tuple(x_ref[i].astype(jnp.float32) for i in range(len(e_refs))))
    for i, h in enumerate(hs):
        o_ref[i] = h


def _pack_weights(packed_w, packed_b, L, C):
    CP = 8 * C
    taps = packed_w.reshape(L, 3, 3, C, C)
    q = jnp.arange(8)
    mask = (q[None, :, None] == q[None, None, :] +
            jnp.arange(3)[:, None, None] - 1).astype(jnp.float32)
    wd = jnp.einsum('xqp,lyxio->lyqipo', mask, taps)
    wd = wd.reshape(L, 3, CP, CP).astype(jnp.bfloat16)
    we = jnp.zeros((L, 3, 2 * C, CP), jnp.float32)
    we = we.at[:, :, 0:C, 0:C].set(taps[:, :, 0])
    we = we.at[:, :, C:2 * C, CP - C:CP].set(taps[:, :, 2])
    we = we.reshape(L, 6 * C, CP).astype(jnp.bfloat16)
    b3 = jnp.tile(packed_b, (1, 8, 1)).reshape(L, 1, CP)
    return wd, we, b3


@jax.jit
def kernel(x, packed_w, packed_b):
    N, H, W, Cin = x.shape
    C = packed_w.shape[-1]
    L = packed_b.shape[0]
    n_sub = L // 7
    S = W // 8
    CP = 8 * C
    ipp = _IPP if N % _IPP == 0 else 1

    wd, we, b3 = _pack_weights(packed_w, packed_b, L, C)
    xp = jnp.pad(x, ((0, 0), (0, 0), (0, 0), (0, C - Cin)))
    xp = xp.reshape(N, H, S, CP).astype(jnp.bfloat16)

    body = functools.partial(_drnn_body, H=H, S=S, CP=CP, n_sub=n_sub)
    out = pl.pallas_call(
        body,
        out_shape=jax.ShapeDtypeStruct((N, H, S, CP), x.dtype),
        grid_spec=pltpu.PrefetchScalarGridSpec(
            num_scalar_prefetch=0,
            grid=(N // ipp,),
            in_specs=[
                pl.BlockSpec((ipp, H, S, CP), lambda n: (n, 0, 0, 0)),
                pl.BlockSpec(wd.shape, lambda n: (0, 0, 0, 0)),
                pl.BlockSpec(we.shape, lambda n: (0, 0, 0)),
                pl.BlockSpec(b3.shape, lambda n: (0, 0, 0)),
            ],
            out_specs=pl.BlockSpec((ipp, H, S, CP), lambda n: (n, 0, 0, 0)),
            scratch_shapes=[
                pltpu.VMEM((H, S, 6 * C), jnp.bfloat16)
                for i in range(ipp)
            ],
        ),
        compiler_params=pltpu.CompilerParams(
            dimension_semantics=("parallel",)),
    )(xp, wd, we, b3)
    return out.reshape(N, H, W, C)[..., :Cin]


# speedup vs baseline: 1.4121x; 1.1920x over previous
import functools

import jax
import jax.numpy as jnp
from jax import lax
from jax.experimental import pallas as pl
from jax.experimental.pallas import tpu as pltpu

_IPP = 4


def _drnn_body(x_ref, wd_ref, we_ref, b_ref, o_ref, *scratch,
               H, S, CP, n_sub):
    C = CP // 8
    M = H * S
    e_refs = scratch

    for r in scratch:
        r[...] = jnp.zeros_like(r)

    zrow = jnp.zeros((1, S, CP), jnp.bfloat16)

    zcol = jnp.zeros((H, 1, C), jnp.bfloat16)
    zrow2 = jnp.zeros((1, S, 2 * C), jnp.bfloat16)

    def conv3x3(h_val, e_ref, li, relu):
        hb = h_val.astype(jnp.bfloat16)
        left = hb[:, 0:S - 1, CP - C:CP]
        right = hb[:, 1:S, 0:C]
        fv = jnp.concatenate([jnp.concatenate([zcol, left], axis=1),
                              jnp.concatenate([right, zcol], axis=1)], axis=2)
        f0 = jnp.concatenate([zrow2, fv[0:H - 1]], axis=0)
        f2 = jnp.concatenate([fv[1:H], zrow2], axis=0)
        ev = jnp.concatenate([f0, fv, f2], axis=2).reshape(M, 6 * C)
        x0 = jnp.concatenate([zrow, hb[0:H - 1]], axis=0).reshape(M, CP)
        x2 = jnp.concatenate([hb[1:H], zrow], axis=0).reshape(M, CP)
        acc = jnp.dot(x0, wd_ref[li, 0], preferred_element_type=jnp.float32)
        acc = acc + jnp.dot(hb.reshape(M, CP), wd_ref[li, 1],
                            preferred_element_type=jnp.float32)
        acc = acc + jnp.dot(x2, wd_ref[li, 2],
                            preferred_element_type=jnp.float32)
        acc = acc + jnp.dot(ev, we_ref[li],
                            preferred_element_type=jnp.float32)
        acc = acc + b_ref[li]
        if relu:
            acc = jnp.maximum(acc, 0.0)
        return acc.reshape(H, S, CP)

    def convs(hs, li, relu):
        return tuple(conv3x3(h, e_refs[i], li, relu)
                     for i, h in enumerate(hs))

    def add(xs, ys):
        return tuple(xv + yv for xv, yv in zip(xs, ys))

    def subnet(si, hs):
        base = si * 7
        a1 = convs(hs, base + 0, True)
        a2 = convs(a1, base + 1, True)
        a3 = convs(a2, base + 2, True)
        a4 = add(convs(a3, base + 3, True), a3)
        a5 = add(convs(a4, base + 4, True), a2)
        a6 = add(convs(a5, base + 5, True), a1)
        return add(convs(a6, base + 6, False), hs)

    hs = lax.fori_loop(
        0, n_sub, subnet,
        tuple(x_ref[i].astype(jnp.float32) for i in range(len(e_refs))))
    for i, h in enumerate(hs):
        o_ref[i] = h


def _pack_weights(packed_w, packed_b, L, C):
    CP = 8 * C
    taps = packed_w.reshape(L, 3, 3, C, C)
    q = jnp.arange(8)
    mask = (q[None, :, None] == q[None, None, :] +
            jnp.arange(3)[:, None, None] - 1).astype(jnp.float32)
    wd = jnp.einsum('xqp,lyxio->lyqipo', mask, taps)
    wd = wd.reshape(L, 3, CP, CP).astype(jnp.bfloat16)
    we = jnp.zeros((L, 3, 2 * C, CP), jnp.float32)
    we = we.at[:, :, 0:C, 0:C].set(taps[:, :, 0])
    we = we.at[:, :, C:2 * C, CP - C:CP].set(taps[:, :, 2])
    we = we.reshape(L, 6 * C, CP).astype(jnp.bfloat16)
    b3 = jnp.tile(packed_b, (1, 8, 1)).reshape(L, 1, CP)
    return wd, we, b3


@jax.jit
def kernel(x, packed_w, packed_b):
    N, H, W, Cin = x.shape
    C = packed_w.shape[-1]
    L = packed_b.shape[0]
    n_sub = L // 7
    S = W // 8
    CP = 8 * C
    ipp = _IPP if N % _IPP == 0 else 1

    wd, we, b3 = _pack_weights(packed_w, packed_b, L, C)
    xp = jnp.pad(x, ((0, 0), (0, 0), (0, 0), (0, C - Cin)))
    xp = xp.reshape(N, H, S, CP).astype(jnp.bfloat16)

    body = functools.partial(_drnn_body, H=H, S=S, CP=CP, n_sub=n_sub)
    out = pl.pallas_call(
        body,
        out_shape=jax.ShapeDtypeStruct((N, H, S, CP), x.dtype),
        grid_spec=pltpu.PrefetchScalarGridSpec(
            num_scalar_prefetch=0,
            grid=(N // ipp,),
            in_specs=[
                pl.BlockSpec((ipp, H, S, CP), lambda n: (n, 0, 0, 0)),
                pl.BlockSpec(wd.shape, lambda n: (0, 0, 0, 0)),
                pl.BlockSpec(we.shape, lambda n: (0, 0, 0)),
                pl.BlockSpec(b3.shape, lambda n: (0, 0, 0)),
            ],
            out_specs=pl.BlockSpec((ipp, H, S, CP), lambda n: (n, 0, 0, 0)),
            scratch_shapes=[
                pltpu.VMEM((H, S, 6 * C), jnp.bfloat16)
                for i in range(ipp)
            ],
        ),
        compiler_params=pltpu.CompilerParams(
            dimension_semantics=("parallel",)),
    )(xp, wd, we, b3)
    return out.reshape(N, H, W, C)[..., :Cin]


# speedup vs baseline: 1.4127x; 1.0005x over previous
import functools

import jax
import jax.numpy as jnp
from jax import lax
from jax.experimental import pallas as pl
from jax.experimental.pallas import tpu as pltpu

_IPP = 4


def _drnn_body(x_ref, wd_ref, we_ref, b_ref, o_ref, *, H, S, CP, n_sub):
    C = CP // 8
    M = H * S
    ipp = x_ref.shape[0]

    zrow = jnp.zeros((1, S, CP), jnp.bfloat16)

    zcol = jnp.zeros((H, 1, C), jnp.bfloat16)
    zrow2 = jnp.zeros((1, S, 2 * C), jnp.bfloat16)

    def conv3x3(h_val, li, relu):
        hb = h_val.astype(jnp.bfloat16)
        left = hb[:, 0:S - 1, CP - C:CP]
        right = hb[:, 1:S, 0:C]
        fv = jnp.concatenate([jnp.concatenate([zcol, left], axis=1),
                              jnp.concatenate([right, zcol], axis=1)], axis=2)
        f0 = jnp.concatenate([zrow2, fv[0:H - 1]], axis=0)
        f2 = jnp.concatenate([fv[1:H], zrow2], axis=0)
        ev = jnp.concatenate([f0, fv, f2], axis=2).reshape(M, 6 * C)
        x0 = jnp.concatenate([zrow, hb[0:H - 1]], axis=0).reshape(M, CP)
        x2 = jnp.concatenate([hb[1:H], zrow], axis=0).reshape(M, CP)
        acc = jnp.dot(x0, wd_ref[li, 0], preferred_element_type=jnp.float32)
        acc = acc + jnp.dot(hb.reshape(M, CP), wd_ref[li, 1],
                            preferred_element_type=jnp.float32)
        acc = acc + jnp.dot(x2, wd_ref[li, 2],
                            preferred_element_type=jnp.float32)
        acc = acc + jnp.dot(ev, we_ref[li],
                            preferred_element_type=jnp.float32)
        acc = acc + b_ref[li]
        if relu:
            acc = jnp.maximum(acc, 0.0)
        return acc.reshape(H, S, CP)

    def convs(hs, li, relu):
        return tuple(conv3x3(h, li, relu) for h in hs)

    def add(xs, ys):
        return tuple(xv + yv for xv, yv in zip(xs, ys))

    def subnet(si, hs):
        base = si * 7
        a1 = convs(hs, base + 0, True)
        a2 = convs(a1, base + 1, True)
        a3 = convs(a2, base + 2, True)
        a4 = add(convs(a3, base + 3, True), a3)
        a5 = add(convs(a4, base + 4, True), a2)
        a6 = add(convs(a5, base + 5, True), a1)
        return add(convs(a6, base + 6, False), hs)

    hs = lax.fori_loop(
        0, n_sub, subnet,
        tuple(x_ref[i].astype(jnp.float32) for i in range(ipp)))
    for i, h in enumerate(hs):
        o_ref[i] = h


def _pack_weights(packed_w, packed_b, L, C):
    CP = 8 * C
    taps = packed_w.reshape(L, 3, 3, C, C)
    q = jnp.arange(8)
    mask = (q[None, :, None] == q[None, None, :] +
            jnp.arange(3)[:, None, None] - 1).astype(jnp.float32)
    wd = jnp.einsum('xqp,lyxio->lyqipo', mask, taps)
    wd = wd.reshape(L, 3, CP, CP).astype(jnp.bfloat16)
    we = jnp.zeros((L, 3, 2 * C, CP), jnp.float32)
    we = we.at[:, :, 0:C, 0:C].set(taps[:, :, 0])
    we = we.at[:, :, C:2 * C, CP - C:CP].set(taps[:, :, 2])
    we = we.reshape(L, 6 * C, CP).astype(jnp.bfloat16)
    b3 = jnp.tile(packed_b, (1, 8, 1)).reshape(L, 1, CP)
    return wd, we, b3


@jax.jit
def kernel(x, packed_w, packed_b):
    N, H, W, Cin = x.shape
    C = packed_w.shape[-1]
    L = packed_b.shape[0]
    n_sub = L // 7
    S = W // 8
    CP = 8 * C
    ipp = _IPP if N % _IPP == 0 else 1

    wd, we, b3 = _pack_weights(packed_w, packed_b, L, C)
    xp = jnp.pad(x, ((0, 0), (0, 0), (0, 0), (0, C - Cin)))
    xp = xp.reshape(N, H, S, CP).astype(jnp.bfloat16)

    body = functools.partial(_drnn_body, H=H, S=S, CP=CP, n_sub=n_sub)
    out = pl.pallas_call(
        body,
        out_shape=jax.ShapeDtypeStruct((N, H, S, CP), x.dtype),
        grid_spec=pltpu.PrefetchScalarGridSpec(
            num_scalar_prefetch=0,
            grid=(N // ipp,),
            in_specs=[
                pl.BlockSpec((ipp, H, S, CP), lambda n: (n, 0, 0, 0)),
                pl.BlockSpec(wd.shape, lambda n: (0, 0, 0, 0)),
                pl.BlockSpec(we.shape, lambda n: (0, 0, 0)),
                pl.BlockSpec(b3.shape, lambda n: (0, 0, 0)),
            ],
            out_specs=pl.BlockSpec((ipp, H, S, CP), lambda n: (n, 0, 0, 0)),
            scratch_shapes=[],
        ),
        compiler_params=pltpu.CompilerParams(
            dimension_semantics=("parallel",)),
    )(xp, wd, we, b3)
    return out.reshape(N, H, W, C)[..., :Cin]
